# Initial kernel scaffold; baseline (speedup 1.0000x reference)
import jax
import jax.numpy as jnp
from jax.experimental import pallas as pl


def kernel(x, table):
    raise NotImplementedError("write your pallas kernel here")


# speedup vs baseline: 1.6849x; 1.6849x over previous
import functools

import jax
import jax.numpy as jnp
from jax import lax
from jax.experimental import pallas as pl
from jax.experimental.pallas import tpu as pltpu
from jax.experimental.pallas import tpu_sc as plsc

NC, NS = 2, 16
NW = NC * NS
CHUNK = 128


@functools.lru_cache(maxsize=None)
def _make_gather(n_rows: int, emb: int):
    assert n_rows % (NW * CHUNK) == 0
    b_per_w = n_rows // NW
    n_chunks = b_per_w // CHUNK
    mesh = plsc.VectorSubcoreMesh(
        core_axis_name="c", subcore_axis_name="s",
        num_cores=NC, num_subcores=NS,
    )

    @functools.partial(
        pl.kernel,
        out_type=jax.ShapeDtypeStruct((n_rows, emb), jnp.float32),
        mesh=mesh,
        scratch_types=[
            pltpu.VMEM((b_per_w,), jnp.int32),
            pltpu.VMEM((CHUNK, emb), jnp.float32),
            pltpu.SemaphoreType.DMA,
        ],
        compiler_params=pltpu.CompilerParams(use_tc_tiling_on_sc=False),
    )
    def gather_kernel(idx_hbm, table_hbm, out_hbm, idx_v, rows_v, sem):
        wid = lax.axis_index("s") * NC + lax.axis_index("c")
        base = wid * b_per_w
        pltpu.sync_copy(idx_hbm.at[pl.ds(base, b_per_w)], idx_v)

        def body(j, carry):
            off = j * CHUNK
            pltpu.async_copy(
                table_hbm.at[idx_v.at[pl.ds(off, CHUNK)]], rows_v, sem
            ).wait()
            pltpu.sync_copy(rows_v, out_hbm.at[pl.ds(base + off, CHUNK)])
            return carry

        lax.fori_loop(0, n_chunks, body, 0)

    return gather_kernel


def kernel(x, table):
    b, l = x.shape
    emb = table.shape[1]
    out = _make_gather(b * l, emb)(x.reshape(-1), table)
    return out.reshape(b, l, emb)


# speedup vs baseline: 1.8765x; 1.1137x over previous
import functools

import jax
import jax.numpy as jnp
from jax import lax
from jax.experimental import pallas as pl
from jax.experimental.pallas import tpu as pltpu
from jax.experimental.pallas import tpu_sc as plsc

NC, NS = 2, 16
NW = NC * NS
CHUNK = 128
KCHUNKS = 4
GROUP = CHUNK * KCHUNKS


@functools.lru_cache(maxsize=None)
def _make_gather(n_rows: int, emb: int):
    assert n_rows % (NW * GROUP * 2) == 0
    b_per_w = n_rows // NW
    n_groups = b_per_w // GROUP
    mesh = plsc.VectorSubcoreMesh(
        core_axis_name="c", subcore_axis_name="s",
        num_cores=NC, num_subcores=NS,
    )

    @functools.partial(
        pl.kernel,
        out_type=jax.ShapeDtypeStruct((n_rows, emb), jnp.float32),
        mesh=mesh,
        scratch_types=[
            pltpu.VMEM((b_per_w,), jnp.int32),
            pltpu.VMEM((GROUP, emb), jnp.float32),
            pltpu.VMEM((GROUP, emb), jnp.float32),
            pltpu.SemaphoreType.DMA,
            pltpu.SemaphoreType.DMA,
        ],
        compiler_params=pltpu.CompilerParams(use_tc_tiling_on_sc=False),
    )
    def gather_kernel(idx_hbm, table_hbm, out_hbm, idx_v, buf0, buf1, sem0, sem1):
        wid = lax.axis_index("s") * NC + lax.axis_index("c")
        base = wid * b_per_w
        pltpu.sync_copy(idx_hbm.at[pl.ds(base, b_per_w)], idx_v)

        def fire(g, buf, sem):
            off = g * GROUP
            for c in range(KCHUNKS):
                pltpu.async_copy(
                    table_hbm.at[idx_v.at[pl.ds(off + c * CHUNK, CHUNK)]],
                    buf.at[pl.ds(c * CHUNK, CHUNK)],
                    sem,
                )

        def drain_write(g, buf, sem):
            pltpu.make_async_copy(table_hbm.at[pl.ds(0, GROUP)], buf, sem).wait()
            pltpu.sync_copy(buf, out_hbm.at[pl.ds(base + g * GROUP, GROUP)])

        fire(0, buf0, sem0)

        def body(i, carry):
            ga = 2 * i
            gb = 2 * i + 1
            fire(gb, buf1, sem1)
            drain_write(ga, buf0, sem0)

            @pl.when(gb + 1 < n_groups)
            def _():
                fire(gb + 1, buf0, sem0)

            drain_write(gb, buf1, sem1)
            return carry

        lax.fori_loop(0, n_groups // 2, body, 0)

    return gather_kernel


def kernel(x, table):
    b, l = x.shape
    emb = table.shape[1]
    out = _make_gather(b * l, emb)(x.reshape(-1), table)
    return out.reshape(b, l, emb)
